# Initial kernel scaffold; baseline (speedup 1.0000x reference)
import jax
import jax.numpy as jnp
from jax.experimental import pallas as pl


def kernel(x, edge_index, batch, W1, b1, W2, b2, lin_W, lin_b, pool_weight, view_att, view_bias, alpha, beta):
    raise NotImplementedError("write your pallas kernel here")


# speedup vs baseline: 48.2383x; 48.2383x over previous
import functools
import math

import jax
import jax.numpy as jnp
from jax import lax
from jax.experimental import pallas as pl
from jax.experimental.pallas import tpu as pltpu
from jax.experimental.pallas import tpu_sc as plsc

N = 10000
NP = 10240
NG = 80
D = 128
E = 320000
CH = 128
NCHUNK = 2560
EP = NCHUNK * CH
NWORK = 32
CHF = 64
CPW_AGG = NCHUNK // NWORK
CPT = NCHUNK // 16
SLICE = NP // 16
K1 = int(math.ceil(0.8 * N))
K2 = int(math.ceil(0.8 * K1))

@functools.lru_cache(maxsize=None)
def _get_mesh():
    return plsc.VectorSubcoreMesh(core_axis_name="c", subcore_axis_name="s")


@functools.lru_cache(maxsize=None)
def _sc_stage_build(n_valid: float):
    n_valid = float(n_valid)

    @functools.partial(
        pl.kernel,
        out_type=[
            jax.ShapeDtypeStruct((NP,), jnp.float32),
            jax.ShapeDtypeStruct((NP,), jnp.float32),
        ],
        mesh=_get_mesh(),
        compiler_params=pltpu.CompilerParams(needs_layout_passes=False),
        scratch_types=[
            pltpu.VMEM((CPT, CH), jnp.int32),
            pltpu.VMEM((CPT, CH), jnp.int32),
            pltpu.VMEM((NP,), jnp.float32),
            pltpu.VMEM((NP,), jnp.float32),
            pltpu.VMEM((CH,), jnp.float32),
            pltpu.VMEM((SLICE,), jnp.float32),
            pltpu.VMEM((SLICE,), jnp.float32),
            pltpu.VMEM((SLICE,), jnp.float32),
            pltpu.VMEM((SLICE,), jnp.float32),
            pltpu.VMEM_SHARED((NP,), jnp.float32),
            pltpu.VMEM_SHARED((NP,), jnp.float32),
            pltpu.VMEM_SHARED((NP,), jnp.float32),
        ],
    )
    def stage_kernel(row_hbm, col_hbm, mask_hbm, deg_out, pr_out,
                     gR, gC, mask_t, q_t, wbuf, sl_q, sl_z, sl_od, sl_tmp,
                     acc_a, acc_p, q_sh):
        c = lax.axis_index("c")
        s = lax.axis_index("s")
        is1 = (c == 1)
        f1 = is1.astype(jnp.float32)
        cbase = s * CPT
        sbase = s * SLICE

        pltpu.sync_copy(row_hbm.at[pl.ds(cbase, CPT)], gR)
        pltpu.sync_copy(col_hbm.at[pl.ds(cbase, CPT)], gC)
        pltpu.sync_copy(mask_hbm, mask_t)

        def zfill(i, _):
            sl_z[pl.ds(i * 16, 16)] = jnp.zeros((16,), jnp.float32)
            return 0
        lax.fori_loop(0, SLICE // 16, zfill, 0)
        pltpu.sync_copy(sl_z, acc_a.at[pl.ds(sbase, SLICE)])
        pltpu.sync_copy(sl_z, acc_p.at[pl.ds(sbase, SLICE)])
        plsc.subcore_barrier()

        def hist_body(j, _):
            for l in range(CH // 16):
                r16 = gR[j, pl.ds(l * 16, 16)]
                c16 = gC[j, pl.ds(l * 16, 16)]
                wc = plsc.load_gather(mask_t, [c16])
                wr = plsc.load_gather(mask_t, [r16])
                w16 = wc * (f1 * wr + (1.0 - f1))
                wbuf[pl.ds(l * 16, 16)] = w16

            @pl.when(jnp.logical_not(is1))
            def _():
                pltpu.sync_copy(wbuf, acc_a.at[gR.at[j]], add=True)

            @pl.when(is1)
            def _():
                pltpu.sync_copy(wbuf, acc_a.at[gC.at[j]], add=True)
            return 0
        lax.fori_loop(0, CPT, hist_body, 0)
        plsc.subcore_barrier()

        @pl.when(jnp.logical_not(is1))
        def _():
            pltpu.sync_copy(acc_a.at[pl.ds(sbase, SLICE)], deg_out.at[pl.ds(sbase, SLICE)])

        @pl.when(is1)
        def _():
            pltpu.sync_copy(acc_a.at[pl.ds(sbase, SLICE)], sl_od)

            def qinit(i, _):
                od = sl_od[pl.ds(i * 16, 16)]
                m = mask_t[pl.ds(sbase + i * 16, 16)]
                sl_q[pl.ds(i * 16, 16)] = m / ((od + 1e-10) * n_valid)
                return 0
            lax.fori_loop(0, SLICE // 16, qinit, 0)
            pltpu.sync_copy(sl_q, q_sh.at[pl.ds(sbase, SLICE)])
        plsc.subcore_barrier()

        @pl.when(is1)
        def _():
            pltpu.sync_copy(q_sh, q_t)

        for t in range(3):
            @pl.when(is1)
            def _():
                def pr_body(j, _):
                    for l in range(CH // 16):
                        c16 = gC[j, pl.ds(l * 16, 16)]
                        wbuf[pl.ds(l * 16, 16)] = plsc.load_gather(q_t, [c16])
                    pltpu.sync_copy(wbuf, acc_p.at[gR.at[j]], add=True)
                    return 0
                lax.fori_loop(0, CPT, pr_body, 0)
            plsc.subcore_barrier()

            if t < 2:
                @pl.when(is1)
                def _():
                    pltpu.sync_copy(acc_p.at[pl.ds(sbase, SLICE)], sl_tmp)

                    def qupd(i, _):
                        a = sl_tmp[pl.ds(i * 16, 16)]
                        m = mask_t[pl.ds(sbase + i * 16, 16)]
                        od = sl_od[pl.ds(i * 16, 16)]
                        p = ((1.0 - 0.85) / n_valid + 0.85 * a) * m
                        sl_q[pl.ds(i * 16, 16)] = p / (od + 1e-10)
                        return 0
                    lax.fori_loop(0, SLICE // 16, qupd, 0)
                    pltpu.sync_copy(sl_q, q_sh.at[pl.ds(sbase, SLICE)])
                    pltpu.sync_copy(sl_z, acc_p.at[pl.ds(sbase, SLICE)])
                plsc.subcore_barrier()

                @pl.when(is1)
                def _():
                    pltpu.sync_copy(q_sh, q_t)
            else:
                @pl.when(is1)
                def _():
                    pltpu.sync_copy(acc_p.at[pl.ds(sbase, SLICE)], pr_out.at[pl.ds(sbase, SLICE)])

    return stage_kernel


def _sc_stage1(row2d, col2d, mask):
    return _sc_stage_build(float(N))(row2d, col2d, mask)


def _sc_stage2(row2d, col2d, mask):
    return _sc_stage_build(float(K1))(row2d, col2d, mask)


@functools.lru_cache(maxsize=None)
def _sc_agg_build():
    NHC = CPW_AGG * 2

    @functools.partial(
        pl.kernel,
        out_type=jax.ShapeDtypeStruct((2, NP, D), jnp.float32),
        mesh=_get_mesh(),
        compiler_params=pltpu.CompilerParams(needs_layout_passes=False),
        scratch_types=[
            pltpu.VMEM((CPW_AGG, CH), jnp.int32),
            pltpu.VMEM((CPW_AGG, CH), jnp.int32),
            pltpu.VMEM((2, CHF), jnp.int32),
            pltpu.VMEM((2, CHF), jnp.int32),
            pltpu.VMEM((2, CHF, D), jnp.float32),
            pltpu.VMEM_SHARED((NP, D), jnp.float32),
            pltpu.SemaphoreType.DMA,
        ],
    )
    def agg_kernel(xwp_hbm, row_hbm, col_hbm, out_hbm, cid, rid, idxg, idxs,
                   rbuf, acc, sem):
        c = lax.axis_index("c")
        s = lax.axis_index("s")
        w = c * 16 + s
        base = w * CPW_AGG
        sbase = s * SLICE

        pltpu.sync_copy(col_hbm.at[pl.ds(base, CPW_AGG)], cid)
        pltpu.sync_copy(row_hbm.at[pl.ds(base, CPW_AGG)], rid)

        def zrow(i, _):
            for l in range(D // 16):
                rbuf[0, i, pl.ds(l * 16, 16)] = jnp.zeros((16,), jnp.float32)
            return 0
        lax.fori_loop(0, CHF, zrow, 0)

        def zcp(i, _):
            pltpu.sync_copy(rbuf.at[0], acc.at[pl.ds(sbase + i * CHF, CHF)])
            return 0
        lax.fori_loop(0, SLICE // CHF, zcp, 0)
        plsc.subcore_barrier()

        def stage_idx(j, slot):
            jj = lax.div(j, 2)
            off = lax.rem(j, 2) * CHF
            for l in range(CHF // 16):
                idxg[slot, pl.ds(l * 16, 16)] = cid[jj, pl.ds(off + l * 16, 16)]

        stage_idx(0, 0)
        pltpu.async_copy(xwp_hbm.at[idxg.at[0]], rbuf.at[0], sem)

        def body(j, _):
            slot = lax.rem(j, 2)
            nxt = lax.rem(j + 1, 2)
            jj = lax.div(j, 2)
            off = lax.rem(j, 2) * CHF
            pltpu.make_async_copy(xwp_hbm.at[idxg.at[slot]], rbuf.at[slot], sem).wait()

            @pl.when(j + 1 < NHC)
            def _():
                stage_idx(j + 1, nxt)
                pltpu.async_copy(xwp_hbm.at[idxg.at[nxt]], rbuf.at[nxt], sem)

            for l in range(CHF // 16):
                idxs[slot, pl.ds(l * 16, 16)] = rid[jj, pl.ds(off + l * 16, 16)]
            pltpu.sync_copy(rbuf.at[slot], acc.at[idxs.at[slot]], add=True)
            return 0
        lax.fori_loop(0, NHC, body, 0)
        plsc.subcore_barrier()

        pltpu.sync_copy(acc.at[pl.ds(sbase, SLICE)], out_hbm.at[c, pl.ds(sbase, SLICE)])

    return agg_kernel


def _sc_agg(xwp, row2d, col2d):
    return _sc_agg_build()(xwp, row2d, col2d)


def _tc_std_matmul(xpad, W1):
    def body(x_ref, w_ref, o_ref):
        xv = x_ref[...]
        rows = lax.broadcasted_iota(jnp.int32, (NP, 1), 0)
        valid = (rows < N).astype(jnp.float32)
        mu = jnp.sum(xv, axis=0, keepdims=True) / float(N)
        dvc = (xv - mu) * valid
        var = jnp.sum(dvc * dvc, axis=0, keepdims=True) / float(N)
        sd = jnp.sqrt(var) + 1e-12
        xs = dvc / sd
        o_ref[...] = jnp.dot(xs, w_ref[...], preferred_element_type=jnp.float32)

    return pl.pallas_call(
        body, out_shape=jax.ShapeDtypeStruct((NP, D), jnp.float32),
    )(xpad, W1)


def _tc_prescale(xw, deg_col, mask_col):
    def body(xw_ref, d_ref, m_ref, o_ref):
        dinv = lax.rsqrt(d_ref[...] * m_ref[...] + 1.0)
        o_ref[...] = xw_ref[...] * dinv

    return pl.pallas_call(
        body, out_shape=jax.ShapeDtypeStruct((NP, D), jnp.float32),
    )(xw, deg_col, mask_col)


def _pool_body(final: bool, n_valid: float, k_sel: int,
               agg_ref, xw_ref, deg_ref, pr_ref, deg_col_ref, mask_ref,
               mask_col_ref, b_ref, pw_ref, wnext_ref, x1_ref, linw_ref,
               linb_ref, va_ref, vb_ref, al_ref, be_ref, *outs):
    f32 = jnp.float32
    rows_col = lax.broadcasted_iota(jnp.int32, (NP, 1), 0)
    valid_col = (rows_col < N).astype(f32)
    ig = lax.broadcasted_iota(jnp.int32, (NG, 128), 0)
    il = lax.broadcasted_iota(jnp.int32, (NG, 128), 1)
    valid_lane = ((ig * 128 + il) < N).astype(f32)

    mask_col = mask_col_ref[...] if final else valid_col
    mask_lane = mask_ref[...].reshape(NG, 128) if final else valid_lane

    dinv_col = lax.rsqrt(deg_col_ref[...] * mask_col + 1.0)
    aggs = agg_ref[0] + agg_ref[1]
    xw = xw_ref[...]
    h = jnp.maximum(aggs * dinv_col + xw * dinv_col * dinv_col + b_ref[...][None, :], 0.0)
    h = h * mask_col

    Amat = ((lax.broadcasted_iota(jnp.int32, (NP, NG), 0) // 128)
            == lax.broadcasted_iota(jnp.int32, (NP, NG), 1)).astype(f32)
    Bmat = ((lax.broadcasted_iota(jnp.int32, (NP, 128), 0) % 128)
            == lax.broadcasted_iota(jnp.int32, (NP, 128), 1)).astype(f32)

    def to_lane(v_col):
        return lax.dot_general(Amat, v_col * Bmat, (((0,), (0,)), ((), ())),
                               preferred_element_type=f32)

    def to_col(s_lane):
        return jnp.sum(jnp.dot(Amat, s_lane, preferred_element_type=f32) * Bmat,
                       axis=1, keepdims=True)

    def sigmoid(z):
        return 1.0 / (1.0 + jnp.exp(-z))

    alpha = al_ref[0]
    beta = be_ref[0]

    deg_lane = deg_ref[...].reshape(NG, 128) * mask_lane
    s1 = sigmoid(alpha * jnp.log(deg_lane + 1e-16) + beta)

    pw = pw_ref[...][0]
    wnorm = jnp.sqrt(jnp.sum(pw * pw))
    rs_col = jnp.sum(h * pw[None, :], axis=1, keepdims=True) / wnorm
    s2 = sigmoid(to_lane(rs_col))

    p3 = ((1.0 - 0.85) / n_valid + 0.85 * pr_ref[...].reshape(NG, 128)) * mask_lane
    s3 = sigmoid(p3)

    sc = [s1 * mask_lane, s2 * mask_lane, s3 * mask_lane]
    scn = [s / jnp.max(s) for s in sc]
    raw = [sigmoid(scn[0] * va_ref[0, j] + scn[1] * va_ref[1, j]
                   + scn[2] * va_ref[2, j] + vb_ref[j]) for j in range(3)]
    rmax = jnp.maximum(jnp.maximum(raw[0], raw[1]), raw[2])
    ex = [jnp.exp(r - rmax) for r in raw]
    esum = ex[0] + ex[1] + ex[2]
    score = sigmoid((scn[0] * ex[0] + scn[1] * ex[1] + scn[2] * ex[2]) / esum)

    s_sel = jnp.where(mask_lane > 0.0, score, -1.0)
    si = lax.bitcast_convert_type(s_sel, jnp.int32)

    def bs(i, t):
        cand = t | lax.shift_left(jnp.int32(1), 30 - i)
        cnt = jnp.sum((si >= cand).astype(jnp.int32))
        return jnp.where(cnt >= k_sel, cand, t)
    T = lax.fori_loop(0, 31, bs, jnp.int32(0))

    ngt = jnp.sum((si > T).astype(jnp.int32))
    need = (k_sel - ngt).astype(f32)
    eq = (si == T).astype(f32)
    U = (lax.broadcasted_iota(jnp.int32, (128, 128), 0)
         < lax.broadcasted_iota(jnp.int32, (128, 128), 1)).astype(f32)
    Lm = (lax.broadcasted_iota(jnp.int32, (NG, NG), 1)
          < lax.broadcasted_iota(jnp.int32, (NG, NG), 0)).astype(f32)
    ones128 = jnp.ones((128, 128), f32)
    r1 = jnp.dot(eq, ones128, preferred_element_type=f32)
    excl = jnp.dot(Lm, r1, preferred_element_type=f32) + \
        jnp.dot(eq, U, preferred_element_type=f32)
    newmask_lane = jnp.where((si > T) | ((eq > 0.0) & (excl < need)), 1.0, 0.0)

    hp = h * to_col(score * newmask_lane)
    gmax = jnp.max(hp, axis=0)
    gmean = jnp.sum(hp, axis=0) / float(k_sel)
    xr = jnp.concatenate([gmax, gmean])[None, :]

    if not final:
        o_xw2, o_mask, o_x1 = outs
        o_xw2[...] = jnp.dot(hp, wnext_ref[...], preferred_element_type=f32)
        o_mask[...] = newmask_lane.reshape(NP)
        o_x1[...] = xr
    else:
        o_out, = outs
        hc = jnp.maximum(x1_ref[...][0], 0.0) + jnp.maximum(xr[0], 0.0)
        ho = jnp.sum(hc[:, None] * linw_ref[...], axis=0) + linb_ref[...]
        ho = jnp.maximum(ho, 0.0)
        m = jnp.max(ho)
        lse = jnp.log(jnp.sum(jnp.exp(ho - m)))
        o_out[...] = (ho - m - lse)[None, :]


def _tc_pool1(agg, xw, deg, pr, deg_col, b1, pw, W2, va, vb, alpha, beta):
    dummy_lane = jnp.zeros((NP,), jnp.float32)
    dummy_col = jnp.zeros((NP, 1), jnp.float32)
    dummy_x1 = jnp.zeros((1, 2 * D), jnp.float32)
    dummy_lw = jnp.zeros((2 * D, 64), jnp.float32)
    dummy_lb = jnp.zeros((64,), jnp.float32)
    body = functools.partial(_pool_body, False, float(N), K1)
    smem = pl.BlockSpec(memory_space=pltpu.SMEM)
    return pl.pallas_call(
        body,
        out_shape=[
            jax.ShapeDtypeStruct((NP, D), jnp.float32),
            jax.ShapeDtypeStruct((NP,), jnp.float32),
            jax.ShapeDtypeStruct((1, 2 * D), jnp.float32),
        ],
        in_specs=[pl.BlockSpec(None)] * 13 + [smem, smem, smem, smem],
    )(agg, xw, deg, pr, deg_col, dummy_lane, dummy_col, b1, pw, W2,
      dummy_x1, dummy_lw, dummy_lb, va, vb, alpha, beta)


def _tc_pool2(agg, xw, deg, pr, deg_col, mask1, mask1_col, b2, pw, x1,
              lin_W, lin_b, va, vb, alpha, beta):
    dummy_w = jnp.zeros((D, D), jnp.float32)
    body = functools.partial(_pool_body, True, float(K1), K2)
    smem = pl.BlockSpec(memory_space=pltpu.SMEM)
    return pl.pallas_call(
        body,
        out_shape=[jax.ShapeDtypeStruct((1, 64), jnp.float32)],
        in_specs=[pl.BlockSpec(None)] * 13 + [smem, smem, smem, smem],
    )(agg, xw, deg, pr, deg_col, mask1, mask1_col, b2, pw, dummy_w,
      x1, lin_W, lin_b, va, vb, alpha, beta)[0]


def kernel(x, edge_index, batch, W1, b1, W2, b2, lin_W, lin_b, pool_weight,
           view_att, view_bias, alpha, beta):
    f32 = jnp.float32
    row = edge_index[0].astype(jnp.int32)
    col = edge_index[1].astype(jnp.int32)
    npad = EP - E
    padidx = (N + (jnp.arange(npad, dtype=jnp.int32) % (NP - N)))
    row2d = jnp.concatenate([row, padidx]).reshape(NCHUNK, CH)
    col2d = jnp.concatenate([col, padidx]).reshape(NCHUNK, CH)

    xpad = jnp.pad(x, ((0, NP - N), (0, 0)))
    mask0 = (jnp.arange(NP) < N).astype(f32)

    xw1 = _tc_std_matmul(xpad, W1)
    deg1, pr1 = _sc_stage1(row2d, col2d, mask0)
    deg1_col = deg1.reshape(NP, 1)
    xwp1 = _tc_prescale(xw1, deg1_col, mask0.reshape(NP, 1))
    agg1 = _sc_agg(xwp1, row2d, col2d)
    xw2, mask1, x1 = _tc_pool1(agg1, xw1, deg1, pr1, deg1_col, b1,
                               pool_weight, W2, view_att, view_bias, alpha, beta)

    deg2, pr2 = _sc_stage2(row2d, col2d, mask1)
    deg2_col = deg2.reshape(NP, 1)
    mask1_col = mask1.reshape(NP, 1)
    xwp2 = _tc_prescale(xw2, deg2_col, mask1_col)
    agg2 = _sc_agg(xwp2, row2d, col2d)
    out = _tc_pool2(agg2, xw2, deg2, pr2, deg2_col, mask1, mask1_col, b2,
                    pool_weight, x1, lin_W, lin_b, view_att, view_bias,
                    alpha, beta)
    return out


# speedup vs baseline: 60.1295x; 1.2465x over previous
import functools
import math

import jax
import jax.numpy as jnp
from jax import lax
from jax.experimental import pallas as pl
from jax.experimental.pallas import tpu as pltpu
from jax.experimental.pallas import tpu_sc as plsc

N = 10000
NP = 10240
NG = 80
D = 128
E = 320000
CH = 128
NCHUNK = 2560
EP = NCHUNK * CH
NWORK = 32
CHF = 64
CPW_AGG = NCHUNK // NWORK
CPT = NCHUNK // 16
SLICE = NP // 16
K1 = int(math.ceil(0.8 * N))
K2 = int(math.ceil(0.8 * K1))

@functools.lru_cache(maxsize=None)
def _get_mesh():
    return plsc.VectorSubcoreMesh(core_axis_name="c", subcore_axis_name="s")


@functools.lru_cache(maxsize=None)
def _sc_stage_build(n_valid: float):
    n_valid = float(n_valid)

    @functools.partial(
        pl.kernel,
        out_type=[
            jax.ShapeDtypeStruct((NP,), jnp.float32),
            jax.ShapeDtypeStruct((NP,), jnp.float32),
        ],
        mesh=_get_mesh(),
        compiler_params=pltpu.CompilerParams(needs_layout_passes=False),
        scratch_types=[
            pltpu.VMEM((CPT, CH), jnp.int32),
            pltpu.VMEM((CPT, CH), jnp.int32),
            pltpu.VMEM((NP,), jnp.float32),
            pltpu.VMEM((NP,), jnp.float32),
            pltpu.VMEM((CH,), jnp.float32),
            pltpu.VMEM((SLICE,), jnp.float32),
            pltpu.VMEM((SLICE,), jnp.float32),
            pltpu.VMEM((SLICE,), jnp.float32),
            pltpu.VMEM((SLICE,), jnp.float32),
            pltpu.VMEM_SHARED((NP,), jnp.float32),
            pltpu.VMEM_SHARED((NP,), jnp.float32),
            pltpu.VMEM_SHARED((NP,), jnp.float32),
        ],
    )
    def stage_kernel(row_hbm, col_hbm, mask_hbm, deg_out, pr_out,
                     gR, gC, mask_t, q_t, wbuf, sl_q, sl_z, sl_od, sl_tmp,
                     acc_a, acc_p, q_sh):
        c = lax.axis_index("c")
        s = lax.axis_index("s")
        is1 = (c == 1)
        f1 = is1.astype(jnp.float32)
        cbase = s * CPT
        sbase = s * SLICE

        pltpu.sync_copy(row_hbm.at[pl.ds(cbase, CPT)], gR)
        pltpu.sync_copy(col_hbm.at[pl.ds(cbase, CPT)], gC)
        pltpu.sync_copy(mask_hbm, mask_t)

        def zfill(i, _):
            sl_z[pl.ds(i * 16, 16)] = jnp.zeros((16,), jnp.float32)
            return 0
        lax.fori_loop(0, SLICE // 16, zfill, 0)
        pltpu.sync_copy(sl_z, acc_a.at[pl.ds(sbase, SLICE)])
        pltpu.sync_copy(sl_z, acc_p.at[pl.ds(sbase, SLICE)])
        plsc.subcore_barrier()

        def hist_body(j, _):
            for l in range(CH // 16):
                r16 = gR[j, pl.ds(l * 16, 16)]
                c16 = gC[j, pl.ds(l * 16, 16)]
                wc = plsc.load_gather(mask_t, [c16])
                wr = plsc.load_gather(mask_t, [r16])
                w16 = wc * (f1 * wr + (1.0 - f1))
                wbuf[pl.ds(l * 16, 16)] = w16

            @pl.when(jnp.logical_not(is1))
            def _():
                pltpu.sync_copy(wbuf, acc_a.at[gR.at[j]], add=True)

            @pl.when(is1)
            def _():
                pltpu.sync_copy(wbuf, acc_a.at[gC.at[j]], add=True)
            return 0
        lax.fori_loop(0, CPT, hist_body, 0)
        plsc.subcore_barrier()

        @pl.when(jnp.logical_not(is1))
        def _():
            pltpu.sync_copy(acc_a.at[pl.ds(sbase, SLICE)], deg_out.at[pl.ds(sbase, SLICE)])

        @pl.when(is1)
        def _():
            pltpu.sync_copy(acc_a.at[pl.ds(sbase, SLICE)], sl_od)

            def qinit(i, _):
                od = sl_od[pl.ds(i * 16, 16)]
                m = mask_t[pl.ds(sbase + i * 16, 16)]
                sl_q[pl.ds(i * 16, 16)] = m / ((od + 1e-10) * n_valid)
                return 0
            lax.fori_loop(0, SLICE // 16, qinit, 0)
            pltpu.sync_copy(sl_q, q_sh.at[pl.ds(sbase, SLICE)])
        plsc.subcore_barrier()

        @pl.when(is1)
        def _():
            pltpu.sync_copy(q_sh, q_t)

        for t in range(3):
            @pl.when(is1)
            def _():
                def pr_body(j, _):
                    for l in range(CH // 16):
                        c16 = gC[j, pl.ds(l * 16, 16)]
                        wbuf[pl.ds(l * 16, 16)] = plsc.load_gather(q_t, [c16])
                    pltpu.sync_copy(wbuf, acc_p.at[gR.at[j]], add=True)
                    return 0
                lax.fori_loop(0, CPT, pr_body, 0)
            plsc.subcore_barrier()

            if t < 2:
                @pl.when(is1)
                def _():
                    pltpu.sync_copy(acc_p.at[pl.ds(sbase, SLICE)], sl_tmp)

                    def qupd(i, _):
                        a = sl_tmp[pl.ds(i * 16, 16)]
                        m = mask_t[pl.ds(sbase + i * 16, 16)]
                        od = sl_od[pl.ds(i * 16, 16)]
                        p = ((1.0 - 0.85) / n_valid + 0.85 * a) * m
                        sl_q[pl.ds(i * 16, 16)] = p / (od + 1e-10)
                        return 0
                    lax.fori_loop(0, SLICE // 16, qupd, 0)
                    pltpu.sync_copy(sl_q, q_sh.at[pl.ds(sbase, SLICE)])
                    pltpu.sync_copy(sl_z, acc_p.at[pl.ds(sbase, SLICE)])
                plsc.subcore_barrier()

                @pl.when(is1)
                def _():
                    pltpu.sync_copy(q_sh, q_t)
            else:
                @pl.when(is1)
                def _():
                    pltpu.sync_copy(acc_p.at[pl.ds(sbase, SLICE)], pr_out.at[pl.ds(sbase, SLICE)])

    return stage_kernel


def _sc_stage1(row2d, col2d, mask):
    return _sc_stage_build(float(N))(row2d, col2d, mask)


def _sc_stage2(row2d, col2d, mask):
    return _sc_stage_build(float(K1))(row2d, col2d, mask)


@functools.lru_cache(maxsize=None)
def _sc_agg_build():
    NHC = CPW_AGG * 2

    @functools.partial(
        pl.kernel,
        out_type=jax.ShapeDtypeStruct((2, NP, D), jnp.float32),
        mesh=_get_mesh(),
        compiler_params=pltpu.CompilerParams(needs_layout_passes=False),
        scratch_types=[
            pltpu.VMEM((CPW_AGG, CH), jnp.int32),
            pltpu.VMEM((CPW_AGG, CH), jnp.int32),
            pltpu.VMEM((3, CHF), jnp.int32),
            pltpu.VMEM((3, CHF), jnp.int32),
            pltpu.VMEM((3, CHF, D), jnp.float32),
            pltpu.VMEM_SHARED((NP, D), jnp.float32),
            pltpu.SemaphoreType.DMA,
            pltpu.SemaphoreType.DMA,
        ],
    )
    def agg_kernel(xwp_hbm, row_hbm, col_hbm, out_hbm, cid, rid, idxg, idxs,
                   rbuf, acc, gsem, ssem):
        c = lax.axis_index("c")
        s = lax.axis_index("s")
        w = c * 16 + s
        base = w * CPW_AGG
        sbase = s * SLICE

        pltpu.sync_copy(col_hbm.at[pl.ds(base, CPW_AGG)], cid)
        pltpu.sync_copy(row_hbm.at[pl.ds(base, CPW_AGG)], rid)

        def zrow(i, _):
            for l in range(D // 16):
                rbuf[0, i, pl.ds(l * 16, 16)] = jnp.zeros((16,), jnp.float32)
            return 0
        lax.fori_loop(0, CHF, zrow, 0)

        def zcp(i, _):
            pltpu.sync_copy(rbuf.at[0], acc.at[pl.ds(sbase + i * CHF, CHF)])
            return 0
        lax.fori_loop(0, SLICE // CHF, zcp, 0)
        plsc.subcore_barrier()

        def stage(dst, slot, src, j):
            jj = lax.div(j, 2)
            off = lax.rem(j, 2) * CHF
            for l in range(CHF // 16):
                dst[slot, pl.ds(l * 16, 16)] = src[jj, pl.ds(off + l * 16, 16)]

        for p in range(3):
            stage(idxg, p, cid, p)
            pltpu.async_copy(xwp_hbm.at[idxg.at[p]], rbuf.at[p], gsem)

        def body(j, _):
            slot = lax.rem(j, 3)
            pltpu.make_async_copy(xwp_hbm.at[idxg.at[slot]], rbuf.at[slot],
                                  gsem).wait()
            stage(idxs, slot, rid, j)
            pltpu.async_copy(rbuf.at[slot], acc.at[idxs.at[slot]], ssem,
                             add=True)

            @pl.when(j >= 1)
            def _():
                prev = lax.rem(j + 2, 3)
                pltpu.make_async_copy(rbuf.at[prev], acc.at[idxs.at[prev]],
                                      ssem).wait()

                @pl.when(j + 2 < NHC)
                def _():
                    stage(idxg, prev, cid, j + 2)
                    pltpu.async_copy(xwp_hbm.at[idxg.at[prev]], rbuf.at[prev],
                                     gsem)
            return 0
        lax.fori_loop(0, NHC, body, 0)
        last = lax.rem(NHC - 1, 3)
        pltpu.make_async_copy(rbuf.at[last], acc.at[idxs.at[last]], ssem).wait()
        plsc.subcore_barrier()

        pltpu.sync_copy(acc.at[pl.ds(sbase, SLICE)], out_hbm.at[c, pl.ds(sbase, SLICE)])

    return agg_kernel


def _sc_agg(xwp, row2d, col2d):
    return _sc_agg_build()(xwp, row2d, col2d)


def _tc_std_matmul(xpad, W1):
    def body(x_ref, w_ref, o_ref):
        xv = x_ref[...]
        rows = lax.broadcasted_iota(jnp.int32, (NP, 1), 0)
        valid = (rows < N).astype(jnp.float32)
        mu = jnp.sum(xv, axis=0, keepdims=True) / float(N)
        dvc = (xv - mu) * valid
        var = jnp.sum(dvc * dvc, axis=0, keepdims=True) / float(N)
        sd = jnp.sqrt(var) + 1e-12
        xs = dvc / sd
        o_ref[...] = jnp.dot(xs, w_ref[...], preferred_element_type=jnp.float32)

    return pl.pallas_call(
        body, out_shape=jax.ShapeDtypeStruct((NP, D), jnp.float32),
    )(xpad, W1)


def _tc_prescale(xw, deg_col, mask_col):
    def body(xw_ref, d_ref, m_ref, o_ref):
        dinv = lax.rsqrt(d_ref[...] * m_ref[...] + 1.0)
        o_ref[...] = xw_ref[...] * dinv

    return pl.pallas_call(
        body, out_shape=jax.ShapeDtypeStruct((NP, D), jnp.float32),
    )(xw, deg_col, mask_col)


def _pool_body(final: bool, n_valid: float, k_sel: int,
               agg_ref, xw_ref, deg_ref, pr_ref, deg_col_ref, mask_ref,
               mask_col_ref, b_ref, pw_ref, wnext_ref, x1_ref, linw_ref,
               linb_ref, va_ref, vb_ref, al_ref, be_ref, *outs):
    f32 = jnp.float32
    rows_col = lax.broadcasted_iota(jnp.int32, (NP, 1), 0)
    valid_col = (rows_col < N).astype(f32)
    ig = lax.broadcasted_iota(jnp.int32, (NG, 128), 0)
    il = lax.broadcasted_iota(jnp.int32, (NG, 128), 1)
    valid_lane = ((ig * 128 + il) < N).astype(f32)

    mask_col = mask_col_ref[...] if final else valid_col
    mask_lane = mask_ref[...].reshape(NG, 128) if final else valid_lane

    dinv_col = lax.rsqrt(deg_col_ref[...] * mask_col + 1.0)
    aggs = agg_ref[0] + agg_ref[1]
    xw = xw_ref[...]
    h = jnp.maximum(aggs * dinv_col + xw * dinv_col * dinv_col + b_ref[...][None, :], 0.0)
    h = h * mask_col

    Amat = ((lax.broadcasted_iota(jnp.int32, (NP, NG), 0) // 128)
            == lax.broadcasted_iota(jnp.int32, (NP, NG), 1)).astype(f32)
    Bmat = ((lax.broadcasted_iota(jnp.int32, (NP, 128), 0) % 128)
            == lax.broadcasted_iota(jnp.int32, (NP, 128), 1)).astype(f32)

    def to_lane(v_col):
        return lax.dot_general(Amat, v_col * Bmat, (((0,), (0,)), ((), ())),
                               preferred_element_type=f32)

    def to_col(s_lane):
        return jnp.sum(jnp.dot(Amat, s_lane, preferred_element_type=f32) * Bmat,
                       axis=1, keepdims=True)

    def sigmoid(z):
        return 1.0 / (1.0 + jnp.exp(-z))

    alpha = al_ref[0]
    beta = be_ref[0]

    deg_lane = deg_ref[...].reshape(NG, 128) * mask_lane
    s1 = sigmoid(alpha * jnp.log(deg_lane + 1e-16) + beta)

    pw = pw_ref[...][0]
    wnorm = jnp.sqrt(jnp.sum(pw * pw))
    rs_col = jnp.sum(h * pw[None, :], axis=1, keepdims=True) / wnorm
    s2 = sigmoid(to_lane(rs_col))

    p3 = ((1.0 - 0.85) / n_valid + 0.85 * pr_ref[...].reshape(NG, 128)) * mask_lane
    s3 = sigmoid(p3)

    sc = [s1 * mask_lane, s2 * mask_lane, s3 * mask_lane]
    scn = [s / jnp.max(s) for s in sc]
    raw = [sigmoid(scn[0] * va_ref[0, j] + scn[1] * va_ref[1, j]
                   + scn[2] * va_ref[2, j] + vb_ref[j]) for j in range(3)]
    rmax = jnp.maximum(jnp.maximum(raw[0], raw[1]), raw[2])
    ex = [jnp.exp(r - rmax) for r in raw]
    esum = ex[0] + ex[1] + ex[2]
    score = sigmoid((scn[0] * ex[0] + scn[1] * ex[1] + scn[2] * ex[2]) / esum)

    s_sel = jnp.where(mask_lane > 0.0, score, -1.0)
    si = lax.bitcast_convert_type(s_sel, jnp.int32)

    def bs(i, t):
        cand = t | lax.shift_left(jnp.int32(1), 30 - i)
        cnt = jnp.sum((si >= cand).astype(jnp.int32))
        return jnp.where(cnt >= k_sel, cand, t)
    T = lax.fori_loop(0, 31, bs, jnp.int32(0))

    ngt = jnp.sum((si > T).astype(jnp.int32))
    need = (k_sel - ngt).astype(f32)
    eq = (si == T).astype(f32)
    U = (lax.broadcasted_iota(jnp.int32, (128, 128), 0)
         < lax.broadcasted_iota(jnp.int32, (128, 128), 1)).astype(f32)
    Lm = (lax.broadcasted_iota(jnp.int32, (NG, NG), 1)
          < lax.broadcasted_iota(jnp.int32, (NG, NG), 0)).astype(f32)
    ones128 = jnp.ones((128, 128), f32)
    r1 = jnp.dot(eq, ones128, preferred_element_type=f32)
    excl = jnp.dot(Lm, r1, preferred_element_type=f32) + \
        jnp.dot(eq, U, preferred_element_type=f32)
    newmask_lane = jnp.where((si > T) | ((eq > 0.0) & (excl < need)), 1.0, 0.0)

    hp = h * to_col(score * newmask_lane)
    gmax = jnp.max(hp, axis=0)
    gmean = jnp.sum(hp, axis=0) / float(k_sel)
    xr = jnp.concatenate([gmax, gmean])[None, :]

    if not final:
        o_xw2, o_mask, o_x1 = outs
        o_xw2[...] = jnp.dot(hp, wnext_ref[...], preferred_element_type=f32)
        o_mask[...] = newmask_lane.reshape(NP)
        o_x1[...] = xr
    else:
        o_out, = outs
        hc = jnp.maximum(x1_ref[...][0], 0.0) + jnp.maximum(xr[0], 0.0)
        ho = jnp.sum(hc[:, None] * linw_ref[...], axis=0) + linb_ref[...]
        ho = jnp.maximum(ho, 0.0)
        m = jnp.max(ho)
        lse = jnp.log(jnp.sum(jnp.exp(ho - m)))
        o_out[...] = (ho - m - lse)[None, :]


def _tc_pool1(agg, xw, deg, pr, deg_col, b1, pw, W2, va, vb, alpha, beta):
    dummy_lane = jnp.zeros((NP,), jnp.float32)
    dummy_col = jnp.zeros((NP, 1), jnp.float32)
    dummy_x1 = jnp.zeros((1, 2 * D), jnp.float32)
    dummy_lw = jnp.zeros((2 * D, 64), jnp.float32)
    dummy_lb = jnp.zeros((64,), jnp.float32)
    body = functools.partial(_pool_body, False, float(N), K1)
    smem = pl.BlockSpec(memory_space=pltpu.SMEM)
    return pl.pallas_call(
        body,
        out_shape=[
            jax.ShapeDtypeStruct((NP, D), jnp.float32),
            jax.ShapeDtypeStruct((NP,), jnp.float32),
            jax.ShapeDtypeStruct((1, 2 * D), jnp.float32),
        ],
        in_specs=[pl.BlockSpec(None)] * 13 + [smem, smem, smem, smem],
    )(agg, xw, deg, pr, deg_col, dummy_lane, dummy_col, b1, pw, W2,
      dummy_x1, dummy_lw, dummy_lb, va, vb, alpha, beta)


def _tc_pool2(agg, xw, deg, pr, deg_col, mask1, mask1_col, b2, pw, x1,
              lin_W, lin_b, va, vb, alpha, beta):
    dummy_w = jnp.zeros((D, D), jnp.float32)
    body = functools.partial(_pool_body, True, float(K1), K2)
    smem = pl.BlockSpec(memory_space=pltpu.SMEM)
    return pl.pallas_call(
        body,
        out_shape=[jax.ShapeDtypeStruct((1, 64), jnp.float32)],
        in_specs=[pl.BlockSpec(None)] * 13 + [smem, smem, smem, smem],
    )(agg, xw, deg, pr, deg_col, mask1, mask1_col, b2, pw, dummy_w,
      x1, lin_W, lin_b, va, vb, alpha, beta)[0]


def kernel(x, edge_index, batch, W1, b1, W2, b2, lin_W, lin_b, pool_weight,
           view_att, view_bias, alpha, beta):
    f32 = jnp.float32
    row = edge_index[0].astype(jnp.int32)
    col = edge_index[1].astype(jnp.int32)
    npad = EP - E
    padidx = (N + (jnp.arange(npad, dtype=jnp.int32) % (NP - N)))
    row2d = jnp.concatenate([row, padidx]).reshape(NCHUNK, CH)
    col2d = jnp.concatenate([col, padidx]).reshape(NCHUNK, CH)

    xpad = jnp.pad(x, ((0, NP - N), (0, 0)))
    mask0 = (jnp.arange(NP) < N).astype(f32)

    xw1 = _tc_std_matmul(xpad, W1)
    deg1, pr1 = _sc_stage1(row2d, col2d, mask0)
    deg1_col = deg1.reshape(NP, 1)
    xwp1 = _tc_prescale(xw1, deg1_col, mask0.reshape(NP, 1))
    agg1 = _sc_agg(xwp1, row2d, col2d)
    xw2, mask1, x1 = _tc_pool1(agg1, xw1, deg1, pr1, deg1_col, b1,
                               pool_weight, W2, view_att, view_bias, alpha, beta)

    deg2, pr2 = _sc_stage2(row2d, col2d, mask1)
    deg2_col = deg2.reshape(NP, 1)
    mask1_col = mask1.reshape(NP, 1)
    xwp2 = _tc_prescale(xw2, deg2_col, mask1_col)
    agg2 = _sc_agg(xwp2, row2d, col2d)
    out = _tc_pool2(agg2, xw2, deg2, pr2, deg2_col, mask1, mask1_col, b2,
                    pool_weight, x1, lin_W, lin_b, view_att, view_bias,
                    alpha, beta)
    return out
